# Initial kernel scaffold; baseline (speedup 1.0000x reference)
import jax
import jax.numpy as jnp
from jax.experimental import pallas as pl


def kernel(x, W, b):
    raise NotImplementedError("write your pallas kernel here")


# speedup vs baseline: 1.8501x; 1.8501x over previous
import functools

import jax
import jax.numpy as jnp
from jax.experimental import pallas as pl

D_MODEL = 768
N_EXP = 64
TEMP = 0.1
BT = 1024


def _router_block(x_ref, wt_ref, b_ref, logits_ref, probs_ref, idx_ref):
    x = x_ref[...]
    logits = (jnp.dot(x, wt_ref[...], preferred_element_type=jnp.float32)
              + b_ref[...][None, :]) / TEMP
    logits_ref[...] = logits

    iota = jax.lax.broadcasted_iota(jnp.int32, logits.shape, 1)
    v1 = jnp.max(logits, axis=1, keepdims=True)
    i1 = jnp.min(jnp.where(logits == v1, iota, N_EXP), axis=1, keepdims=True)
    masked = jnp.where(iota == i1, -jnp.inf, logits)
    v2 = jnp.max(masked, axis=1, keepdims=True)
    i2 = jnp.min(jnp.where(masked == v2, iota, N_EXP), axis=1, keepdims=True)

    e = jnp.exp(v2 - v1)
    p1 = 1.0 / (1.0 + e)
    p2 = e * p1
    probs_ref[...] = jnp.concatenate([p1, p2], axis=1)
    idx_ref[...] = jnp.concatenate([i1, i2], axis=1)


@jax.jit
def kernel(x, W, b):
    n_tokens = x.shape[0]
    grid = (n_tokens // BT,)
    wt = W.T
    logits, probs, idx = pl.pallas_call(
        _router_block,
        grid=grid,
        in_specs=[
            pl.BlockSpec((BT, D_MODEL), lambda i: (i, 0)),
            pl.BlockSpec((D_MODEL, N_EXP), lambda i: (0, 0)),
            pl.BlockSpec((N_EXP,), lambda i: (0,)),
        ],
        out_specs=[
            pl.BlockSpec((BT, N_EXP), lambda i: (i, 0)),
            pl.BlockSpec((BT, 2), lambda i: (i, 0)),
            pl.BlockSpec((BT, 2), lambda i: (i, 0)),
        ],
        out_shape=[
            jax.ShapeDtypeStruct((n_tokens, N_EXP), jnp.float32),
            jax.ShapeDtypeStruct((n_tokens, 2), jnp.float32),
            jax.ShapeDtypeStruct((n_tokens, 2), jnp.int32),
        ],
    )(x, wt, b)
    return (logits, probs, idx)


# speedup vs baseline: 2.0452x; 1.1054x over previous
import functools

import jax
import jax.numpy as jnp
from jax.experimental import pallas as pl

D_MODEL = 768
N_EXP = 64
TEMP = 0.1
BT = 2048


def _router_block(x_ref, wt_ref, b_ref, logits_ref, probs_ref, idx_ref):
    x = x_ref[...]
    logits = (jnp.dot(x, wt_ref[...], preferred_element_type=jnp.float32)
              + b_ref[...][None, :]) / TEMP
    logits_ref[...] = logits

    iota = jax.lax.broadcasted_iota(jnp.int32, logits.shape, 1)
    v1 = jnp.max(logits, axis=1, keepdims=True)
    i1 = jnp.min(jnp.where(logits == v1, iota, N_EXP), axis=1, keepdims=True)
    masked = jnp.where(iota == i1, -jnp.inf, logits)
    v2 = jnp.max(masked, axis=1, keepdims=True)
    i2 = jnp.min(jnp.where(masked == v2, iota, N_EXP), axis=1, keepdims=True)

    e = jnp.exp(v2 - v1)
    p1 = 1.0 / (1.0 + e)
    p2 = e * p1
    probs_ref[...] = jnp.concatenate([p1, p2], axis=1)
    idx_ref[...] = jnp.concatenate([i1, i2], axis=1)


@jax.jit
def kernel(x, W, b):
    n_tokens = x.shape[0]
    grid = (n_tokens // BT,)
    wt = W.T
    logits, probs, idx = pl.pallas_call(
        _router_block,
        grid=grid,
        in_specs=[
            pl.BlockSpec((BT, D_MODEL), lambda i: (i, 0)),
            pl.BlockSpec((D_MODEL, N_EXP), lambda i: (0, 0)),
            pl.BlockSpec((N_EXP,), lambda i: (0,)),
        ],
        out_specs=[
            pl.BlockSpec((BT, N_EXP), lambda i: (i, 0)),
            pl.BlockSpec((BT, 2), lambda i: (i, 0)),
            pl.BlockSpec((BT, 2), lambda i: (i, 0)),
        ],
        out_shape=[
            jax.ShapeDtypeStruct((n_tokens, N_EXP), jnp.float32),
            jax.ShapeDtypeStruct((n_tokens, 2), jnp.float32),
            jax.ShapeDtypeStruct((n_tokens, 2), jnp.int32),
        ],
    )(x, wt, b)
    return (logits, probs, idx)


# speedup vs baseline: 2.1353x; 1.0441x over previous
import functools

import jax
import jax.numpy as jnp
from jax.experimental import pallas as pl

D_MODEL = 768
N_EXP = 64
TEMP = 0.1
BT = 4096


def _router_block(x_ref, wt_ref, b_ref, logits_ref, probs_ref, idx_ref):
    x = x_ref[...]
    logits = (jnp.dot(x, wt_ref[...], preferred_element_type=jnp.float32)
              + b_ref[...][None, :]) / TEMP
    logits_ref[...] = logits

    iota = jax.lax.broadcasted_iota(jnp.int32, logits.shape, 1)
    v1 = jnp.max(logits, axis=1, keepdims=True)
    i1 = jnp.min(jnp.where(logits == v1, iota, N_EXP), axis=1, keepdims=True)
    masked = jnp.where(iota == i1, -jnp.inf, logits)
    v2 = jnp.max(masked, axis=1, keepdims=True)
    i2 = jnp.min(jnp.where(masked == v2, iota, N_EXP), axis=1, keepdims=True)

    e = jnp.exp(v2 - v1)
    p1 = 1.0 / (1.0 + e)
    p2 = e * p1
    probs_ref[...] = jnp.concatenate([p1, p2], axis=1)
    idx_ref[...] = jnp.concatenate([i1, i2], axis=1)


@jax.jit
def kernel(x, W, b):
    n_tokens = x.shape[0]
    grid = (n_tokens // BT,)
    wt = W.T
    logits, probs, idx = pl.pallas_call(
        _router_block,
        grid=grid,
        in_specs=[
            pl.BlockSpec((BT, D_MODEL), lambda i: (i, 0)),
            pl.BlockSpec((D_MODEL, N_EXP), lambda i: (0, 0)),
            pl.BlockSpec((N_EXP,), lambda i: (0,)),
        ],
        out_specs=[
            pl.BlockSpec((BT, N_EXP), lambda i: (i, 0)),
            pl.BlockSpec((BT, 2), lambda i: (i, 0)),
            pl.BlockSpec((BT, 2), lambda i: (i, 0)),
        ],
        out_shape=[
            jax.ShapeDtypeStruct((n_tokens, N_EXP), jnp.float32),
            jax.ShapeDtypeStruct((n_tokens, 2), jnp.float32),
            jax.ShapeDtypeStruct((n_tokens, 2), jnp.int32),
        ],
    )(x, wt, b)
    return (logits, probs, idx)


# speedup vs baseline: 2.4895x; 1.1659x over previous
import functools

import jax
import jax.numpy as jnp
from jax import lax
from jax.experimental import pallas as pl
from jax.experimental.pallas import tpu as pltpu
from jax.experimental.pallas import tpu_sc as plsc

D_MODEL = 768
N_EXP = 64
TEMP = 0.1
N_TOK = 32768
BT = 4096

_info = plsc.get_sparse_core_info()
_NC, _NS, _L = _info.num_cores, _info.num_subcores, _info.num_lanes
_NW = _NC * _NS
TOK_W = N_TOK // _NW
_GRP = TOK_W // _L
_UNROLL = 4


def _logits_block(x_ref, wt_ref, b_ref, logits_ref, logits_t_ref):
    logits = (
        jnp.dot(x_ref[...], wt_ref[...], preferred_element_type=jnp.float32)
        + b_ref[...][None, :]) / TEMP
    logits_ref[...] = logits
    logits_t_ref[...] = logits.T


_sc_mesh = plsc.VectorSubcoreMesh(core_axis_name="c", subcore_axis_name="s")


@functools.partial(
    pl.kernel,
    mesh=_sc_mesh,
    out_type=[
        jax.ShapeDtypeStruct((2, N_TOK), jnp.float32),
        jax.ShapeDtypeStruct((2, N_TOK), jnp.int32),
    ],
    scratch_types=[
        pltpu.VMEM((N_EXP, TOK_W), jnp.float32),
        pltpu.VMEM((TOK_W,), jnp.float32),
        pltpu.VMEM((TOK_W,), jnp.float32),
        pltpu.VMEM((TOK_W,), jnp.int32),
        pltpu.VMEM((TOK_W,), jnp.int32),
    ],
)
def _sc_topk(logits_t_hbm, probs_hbm, idx_hbm, lt_v, p1_v, p2_v, i1_v, i2_v):
    wid = lax.axis_index("s") * _NC + lax.axis_index("c")
    base = wid * TOK_W
    pltpu.sync_copy(logits_t_hbm.at[:, pl.ds(base, TOK_W)], lt_v)

    neg = jnp.full((_L,), -jnp.inf, jnp.float32)
    zero = jnp.zeros((_L,), jnp.int32)

    def super_group(sg, _):
        offs = [sg * (_UNROLL * _L) + g * _L for g in range(_UNROLL)]
        m1 = [neg] * _UNROLL
        m2 = [neg] * _UNROLL
        j1 = [zero] * _UNROLL
        j2 = [zero] * _UNROLL
        for e in range(N_EXP):
            ei = jnp.full((_L,), e, jnp.int32)
            for g in range(_UNROLL):
                v = lt_v[e, pl.ds(offs[g], _L)]
                gt1 = v > m1[g]
                lose = jnp.minimum(v, m1[g])
                gt2 = lose > m2[g]
                nj1 = jnp.where(gt1, ei, j1[g])
                tj = jnp.where(gt1, j1[g], ei)
                j2[g] = jnp.where(gt2, tj, j2[g])
                m1[g] = jnp.maximum(v, m1[g])
                m2[g] = jnp.maximum(lose, m2[g])
                j1[g] = nj1
        for g in range(_UNROLL):
            e2 = jnp.exp(m2[g] - m1[g])
            p1 = 1.0 / (1.0 + e2)
            p1_v[pl.ds(offs[g], _L)] = p1
            p2_v[pl.ds(offs[g], _L)] = e2 * p1
            i1_v[pl.ds(offs[g], _L)] = j1[g]
            i2_v[pl.ds(offs[g], _L)] = j2[g]
        return 0

    lax.fori_loop(0, _GRP // _UNROLL, super_group, 0)

    pltpu.sync_copy(p1_v, probs_hbm.at[0, pl.ds(base, TOK_W)])
    pltpu.sync_copy(p2_v, probs_hbm.at[1, pl.ds(base, TOK_W)])
    pltpu.sync_copy(i1_v, idx_hbm.at[0, pl.ds(base, TOK_W)])
    pltpu.sync_copy(i2_v, idx_hbm.at[1, pl.ds(base, TOK_W)])


@jax.jit
def kernel(x, W, b):
    n_tokens = x.shape[0]
    wt = W.T
    logits, logits_t = pl.pallas_call(
        _logits_block,
        grid=(n_tokens // BT,),
        in_specs=[
            pl.BlockSpec((BT, D_MODEL), lambda i: (i, 0)),
            pl.BlockSpec((D_MODEL, N_EXP), lambda i: (0, 0)),
            pl.BlockSpec((N_EXP,), lambda i: (0,)),
        ],
        out_specs=[
            pl.BlockSpec((BT, N_EXP), lambda i: (i, 0)),
            pl.BlockSpec((N_EXP, BT), lambda i: (0, i)),
        ],
        out_shape=[
            jax.ShapeDtypeStruct((n_tokens, N_EXP), jnp.float32),
            jax.ShapeDtypeStruct((N_EXP, n_tokens), jnp.float32),
        ],
    )(x, wt, b)
    probs_t, idx_t = _sc_topk(logits_t)
    return (logits, probs_t.T, idx_t.T)
